# baseline (speedup 1.0000x reference)
import functools

import jax
import jax.numpy as jnp
from jax.experimental import pallas as pl
from jax.experimental.pallas import tpu as pltpu

N_VARS = 400
HIDDEN = 64
LAYERS = 4
HORIZON = 24
B = 4
L = 36
BN = B * N_VARS
HA = HIDDEN + 8


def _model_kernel(xa_ref, A_ref, W0a_ref, WxrTa_ref, WhT_ref,
                  mpWT_ref, mpbT_ref, WoutT_ref, boutT_ref,
                  out_ref, hsA_ref, hsB_ref, hT_ref, adj_ref):
    f32 = jnp.float32
    H = HIDDEN

    a = A_ref[:, :]
    a = a - jnp.max(a, axis=1, keepdims=True)
    e = jnp.exp(a)
    adj_ref[:, :] = e / jnp.sum(e, axis=1, keepdims=True)

    ones_block = jnp.concatenate(
        [jnp.ones((1, BN), f32), jnp.zeros((7, BN), f32)], axis=0)
    for t in range(L):
        hsA_ref[t, H:, :] = ones_block
        hsB_ref[t, H:, :] = ones_block

    def sigmoid(v):
        return 0.5 * jnp.tanh(0.5 * v) + 0.5

    def gates(gx, gh, hv):
        r = sigmoid(gx[:H] + gh[:H])
        z = sigmoid(gx[H:2 * H] + gh[H:2 * H])
        n = jnp.tanh(gx[2 * H:] + r * gh[2 * H:])
        return n + z * (hv - n)

    def message_passing(l):
        adjv = adj_ref[:, :]
        mpWT_l = mpWT_ref[l]
        mpbT_l = mpbT_ref[l]
        for bb in range(B):
            sl = slice(bb * N_VARS, (bb + 1) * N_VARS)
            hb = hT_ref[:, sl]
            m = jnp.dot(hb, adjv, preferred_element_type=f32)
            mb = jnp.dot(mpWT_l, m, preferred_element_type=f32) + mpbT_l
            elu = jnp.where(mb > 0, mb, jnp.exp(mb) - 1.0)
            hT_ref[:, sl] = elu + hb

    CHUNKS = ((0, 384), (384, 768), (768, 1152), (1152, 1600))

    W0a = W0a_ref[:, :]
    WhT0 = WhT_ref[0]
    hvs = [jnp.zeros((H, c1 - c0), dtype=f32) for c0, c1 in CHUNKS]
    for t in range(L):
        for i, (c0, c1) in enumerate(CHUNKS):
            gx = jnp.dot(W0a, xa_ref[t, :, c0:c1],
                         preferred_element_type=f32)
            gh = jnp.dot(WhT0, hvs[i], preferred_element_type=f32)
            hvs[i] = gates(gx, gh, hvs[i])
            hsA_ref[t, :H, c0:c1] = hvs[i]
    for i, (c0, c1) in enumerate(CHUNKS):
        hT_ref[:, c0:c1] = hvs[i]
    message_passing(0)

    for l in range(1, LAYERS):
        WxrTa_l = WxrTa_ref[l - 1]
        WhT_l = WhT_ref[l]
        src_ref = hsA_ref if l % 2 == 1 else hsB_ref
        dst_ref = hsB_ref if l % 2 == 1 else hsA_ref

        def step(t, hvt, WxrTa_l=WxrTa_l, WhT_l=WhT_l,
                 src_ref=src_ref, dst_ref=dst_ref):
            out = []
            for i, (c0, c1) in enumerate(CHUNKS):
                x_t = src_ref[t, :, c0:c1]
                gx = jnp.dot(WxrTa_l, x_t, preferred_element_type=f32)
                gh = jnp.dot(WhT_l, hvt[i], preferred_element_type=f32)
                hn = gates(gx, gh, hvt[i])
                dst_ref[t, :H, c0:c1] = hn
                out.append(hn)
            return tuple(out)

        hvt = jax.lax.fori_loop(
            0, L, step,
            tuple(hT_ref[:, c0:c1] for c0, c1 in CHUNKS), unroll=2)
        for i, (c0, c1) in enumerate(CHUNKS):
            hT_ref[:, c0:c1] = hvt[i]
        message_passing(l)

    out_ref[:, :] = (jnp.dot(WoutT_ref[:, :], hT_ref[:, :],
                             preferred_element_type=f32) + boutT_ref[:, :])


@functools.partial(jax.jit, static_argnames=())
def kernel(x, adjacency_matrix, gru_Wx_first, gru_Wx_rest, gru_Wh, gru_b,
           mp_W, mp_b, W_out, b_out):
    f32 = jnp.float32
    xrows = x.transpose(1, 0, 2).reshape(L, 1, BN)
    xa = jnp.concatenate(
        [xrows,
         jnp.ones((L, 1, BN), f32),
         jnp.zeros((L, 6, BN), f32)], axis=1)
    W0a = jnp.concatenate(
        [gru_Wx_first.T, gru_b[0][:, None],
         jnp.zeros((3 * HIDDEN, 6), f32)], axis=1)
    WxrTa = jnp.concatenate(
        [gru_Wx_rest.transpose(0, 2, 1),
         gru_b[1:][:, :, None],
         jnp.zeros((LAYERS - 1, 3 * HIDDEN, 7), f32)], axis=2)
    WhT = gru_Wh.transpose(0, 2, 1)
    mpWT = mp_W.transpose(0, 2, 1)
    mpbT = mp_b[:, :, None]
    WoutT = W_out.T
    boutT = b_out[:, None]

    out = pl.pallas_call(
        _model_kernel,
        out_shape=jax.ShapeDtypeStruct((HORIZON, BN), jnp.float32),
        scratch_shapes=[
            pltpu.VMEM((L, HA, BN), jnp.float32),
            pltpu.VMEM((L, HA, BN), jnp.float32),
            pltpu.VMEM((HIDDEN, BN), jnp.float32),
            pltpu.VMEM((N_VARS, N_VARS), jnp.float32),
        ],
        compiler_params=pltpu.CompilerParams(
            vmem_limit_bytes=64 * 1024 * 1024,
        ),
    )(xa, adjacency_matrix, W0a, WxrTa, WhT, mpWT, mpbT, WoutT, boutT)

    return out.reshape(HORIZON, B, N_VARS).transpose(1, 0, 2)


# speedup vs baseline: 1.1059x; 1.1059x over previous
import functools

import jax
import jax.numpy as jnp
from jax.experimental import pallas as pl
from jax.experimental.pallas import tpu as pltpu

N_VARS = 400
HIDDEN = 64
LAYERS = 4
HORIZON = 24
B = 4
L = 36
BN = B * N_VARS
HA = HIDDEN + 8


def _model_kernel(xa_ref, A_ref, W0a_ref, WxrTa_ref, WhT_ref,
                  mpWT_ref, mpbT_ref, WoutT_ref, boutT_ref,
                  out_ref, hsA_ref, hsB_ref, hT_ref, adj_ref):
    f32 = jnp.float32
    H = HIDDEN

    a = A_ref[:, :]
    a = a - jnp.max(a, axis=1, keepdims=True)
    e = jnp.exp(a)
    adj_ref[:, :] = e / jnp.sum(e, axis=1, keepdims=True)

    ones_block = jnp.concatenate(
        [jnp.ones((1, BN), f32), jnp.zeros((7, BN), f32)], axis=0)
    for t in range(L):
        hsA_ref[t, H:, :] = ones_block
        hsB_ref[t, H:, :] = ones_block

    def sigmoid(v):
        return 0.5 * jnp.tanh(0.5 * v) + 0.5

    def gates(gx, gh, hv):
        r = sigmoid(gx[:H] + gh[:H])
        z = sigmoid(gx[H:2 * H] + gh[H:2 * H])
        n = jnp.tanh(gx[2 * H:] + r * gh[2 * H:])
        return n + z * (hv - n)

    def message_passing(l):
        adjv = adj_ref[:, :]
        mpWT_l = mpWT_ref[l]
        mpbT_l = mpbT_ref[l]
        for bb in range(B):
            sl = slice(bb * N_VARS, (bb + 1) * N_VARS)
            hb = hT_ref[:, sl]
            m = jnp.dot(hb, adjv, preferred_element_type=f32)
            mb = jnp.dot(mpWT_l, m, preferred_element_type=f32) + mpbT_l
            elu = jnp.where(mb > 0, mb, jnp.exp(mb) - 1.0)
            hT_ref[:, sl] = elu + hb

    CHUNKS = ((0, 768), (768, 1600))

    W0a = W0a_ref[:, :]
    WhT0 = WhT_ref[0]
    hvs = [jnp.zeros((H, c1 - c0), dtype=f32) for c0, c1 in CHUNKS]
    for t in range(L):
        for i, (c0, c1) in enumerate(CHUNKS):
            gx = jnp.dot(W0a, xa_ref[t, :, c0:c1],
                         preferred_element_type=f32)
            gh = jnp.dot(WhT0, hvs[i], preferred_element_type=f32)
            hvs[i] = gates(gx, gh, hvs[i])
            hsA_ref[t, :H, c0:c1] = hvs[i]
    for i, (c0, c1) in enumerate(CHUNKS):
        hT_ref[:, c0:c1] = hvs[i]
    message_passing(0)

    for l in range(1, LAYERS):
        WxrTa_l = WxrTa_ref[l - 1]
        WhT_l = WhT_ref[l]
        src_ref = hsA_ref if l % 2 == 1 else hsB_ref
        dst_ref = hsB_ref if l % 2 == 1 else hsA_ref

        def step(t, hvt, WxrTa_l=WxrTa_l, WhT_l=WhT_l,
                 src_ref=src_ref, dst_ref=dst_ref):
            out = []
            for i, (c0, c1) in enumerate(CHUNKS):
                x_t = src_ref[t, :, c0:c1]
                gx = jnp.dot(WxrTa_l, x_t, preferred_element_type=f32)
                gh = jnp.dot(WhT_l, hvt[i], preferred_element_type=f32)
                hn = gates(gx, gh, hvt[i])
                dst_ref[t, :H, c0:c1] = hn
                out.append(hn)
            return tuple(out)

        hvt = jax.lax.fori_loop(
            0, L, step,
            tuple(hT_ref[:, c0:c1] for c0, c1 in CHUNKS), unroll=6)
        for i, (c0, c1) in enumerate(CHUNKS):
            hT_ref[:, c0:c1] = hvt[i]
        message_passing(l)

    out_ref[:, :] = (jnp.dot(WoutT_ref[:, :], hT_ref[:, :],
                             preferred_element_type=f32) + boutT_ref[:, :])


@functools.partial(jax.jit, static_argnames=())
def kernel(x, adjacency_matrix, gru_Wx_first, gru_Wx_rest, gru_Wh, gru_b,
           mp_W, mp_b, W_out, b_out):
    f32 = jnp.float32
    xrows = x.transpose(1, 0, 2).reshape(L, 1, BN)
    xa = jnp.concatenate(
        [xrows,
         jnp.ones((L, 1, BN), f32),
         jnp.zeros((L, 6, BN), f32)], axis=1)
    W0a = jnp.concatenate(
        [gru_Wx_first.T, gru_b[0][:, None],
         jnp.zeros((3 * HIDDEN, 6), f32)], axis=1)
    WxrTa = jnp.concatenate(
        [gru_Wx_rest.transpose(0, 2, 1),
         gru_b[1:][:, :, None],
         jnp.zeros((LAYERS - 1, 3 * HIDDEN, 7), f32)], axis=2)
    WhT = gru_Wh.transpose(0, 2, 1)
    mpWT = mp_W.transpose(0, 2, 1)
    mpbT = mp_b[:, :, None]
    WoutT = W_out.T
    boutT = b_out[:, None]

    out = pl.pallas_call(
        _model_kernel,
        out_shape=jax.ShapeDtypeStruct((HORIZON, BN), jnp.float32),
        scratch_shapes=[
            pltpu.VMEM((L, HA, BN), jnp.float32),
            pltpu.VMEM((L, HA, BN), jnp.float32),
            pltpu.VMEM((HIDDEN, BN), jnp.float32),
            pltpu.VMEM((N_VARS, N_VARS), jnp.float32),
        ],
        compiler_params=pltpu.CompilerParams(
            vmem_limit_bytes=64 * 1024 * 1024,
        ),
    )(xa, adjacency_matrix, W0a, WxrTa, WhT, mpWT, mpbT, WoutT, boutT)

    return out.reshape(HORIZON, B, N_VARS).transpose(1, 0, 2)


# speedup vs baseline: 1.1273x; 1.0194x over previous
import functools

import jax
import jax.numpy as jnp
from jax.experimental import pallas as pl
from jax.experimental.pallas import tpu as pltpu

N_VARS = 400
HIDDEN = 64
LAYERS = 4
HORIZON = 24
B = 4
L = 36
BN = B * N_VARS
HA = HIDDEN + 8


def _model_kernel(xa_ref, A_ref, W0a_ref, WxrTa_ref, WhT_ref,
                  mpWT_ref, mpbT_ref, WoutT_ref, boutT_ref,
                  out_ref, hsA_ref, hsB_ref, hT_ref, adj_ref):
    f32 = jnp.float32
    H = HIDDEN

    a = A_ref[:, :]
    a = a - jnp.max(a, axis=1, keepdims=True)
    e = jnp.exp(a)
    adj_ref[:, :] = e / jnp.sum(e, axis=1, keepdims=True)

    ones_block = jnp.concatenate(
        [jnp.ones((1, BN), f32), jnp.zeros((7, BN), f32)], axis=0)
    for t in range(L):
        hsA_ref[t, H:, :] = ones_block
        hsB_ref[t, H:, :] = ones_block

    def sigmoid(v):
        return 0.5 * jnp.tanh(0.5 * v) + 0.5

    def gates(gx, gh, hv):
        r = sigmoid(gx[:H] + gh[:H])
        z = sigmoid(gx[H:2 * H] + gh[H:2 * H])
        n = jnp.tanh(gx[2 * H:] + r * gh[2 * H:])
        return n + z * (hv - n)

    def message_passing(l):
        adjv = adj_ref[:, :]
        mpWT_l = mpWT_ref[l]
        mpbT_l = mpbT_ref[l]
        for bb in range(B):
            sl = slice(bb * N_VARS, (bb + 1) * N_VARS)
            hb = hT_ref[:, sl]
            m = jnp.dot(hb, adjv, preferred_element_type=f32)
            mb = jnp.dot(mpWT_l, m, preferred_element_type=f32) + mpbT_l
            elu = jnp.where(mb > 0, mb, jnp.exp(mb) - 1.0)
            hT_ref[:, sl] = elu + hb

    CHUNKS = ((0, 768), (768, 1600))

    W0a = W0a_ref[:, :]
    WhT0 = WhT_ref[0]
    hvs = [jnp.zeros((H, c1 - c0), dtype=f32) for c0, c1 in CHUNKS]
    for t in range(L):
        for i, (c0, c1) in enumerate(CHUNKS):
            gx = jnp.dot(W0a, xa_ref[t, :, c0:c1],
                         preferred_element_type=f32)
            gh = jnp.dot(WhT0, hvs[i], preferred_element_type=f32)
            hvs[i] = gates(gx, gh, hvs[i])
            hsA_ref[t, :H, c0:c1] = hvs[i]
    for i, (c0, c1) in enumerate(CHUNKS):
        hT_ref[:, c0:c1] = hvs[i]
    message_passing(0)

    for l in range(1, LAYERS):
        WxrTa_l = WxrTa_ref[l - 1]
        WhT_l = WhT_ref[l]
        src_ref = hsA_ref if l % 2 == 1 else hsB_ref
        dst_ref = hsB_ref if l % 2 == 1 else hsA_ref

        def step(t, hvt, WxrTa_l=WxrTa_l, WhT_l=WhT_l,
                 src_ref=src_ref, dst_ref=dst_ref):
            out = []
            for i, (c0, c1) in enumerate(CHUNKS):
                x_t = src_ref[t, :, c0:c1]
                gx = jnp.dot(WxrTa_l, x_t, preferred_element_type=f32)
                gh = jnp.dot(WhT_l, hvt[i], preferred_element_type=f32)
                hn = gates(gx, gh, hvt[i])
                dst_ref[t, :H, c0:c1] = hn
                out.append(hn)
            return tuple(out)

        hvt = jax.lax.fori_loop(
            0, L, step,
            tuple(hT_ref[:, c0:c1] for c0, c1 in CHUNKS), unroll=9)
        for i, (c0, c1) in enumerate(CHUNKS):
            hT_ref[:, c0:c1] = hvt[i]
        message_passing(l)

    out_ref[:, :] = (jnp.dot(WoutT_ref[:, :], hT_ref[:, :],
                             preferred_element_type=f32) + boutT_ref[:, :])


@functools.partial(jax.jit, static_argnames=())
def kernel(x, adjacency_matrix, gru_Wx_first, gru_Wx_rest, gru_Wh, gru_b,
           mp_W, mp_b, W_out, b_out):
    f32 = jnp.float32
    xrows = x.transpose(1, 0, 2).reshape(L, 1, BN)
    xa = jnp.concatenate(
        [xrows,
         jnp.ones((L, 1, BN), f32),
         jnp.zeros((L, 6, BN), f32)], axis=1)
    W0a = jnp.concatenate(
        [gru_Wx_first.T, gru_b[0][:, None],
         jnp.zeros((3 * HIDDEN, 6), f32)], axis=1)
    WxrTa = jnp.concatenate(
        [gru_Wx_rest.transpose(0, 2, 1),
         gru_b[1:][:, :, None],
         jnp.zeros((LAYERS - 1, 3 * HIDDEN, 7), f32)], axis=2)
    WhT = gru_Wh.transpose(0, 2, 1)
    mpWT = mp_W.transpose(0, 2, 1)
    mpbT = mp_b[:, :, None]
    WoutT = W_out.T
    boutT = b_out[:, None]

    out = pl.pallas_call(
        _model_kernel,
        out_shape=jax.ShapeDtypeStruct((HORIZON, BN), jnp.float32),
        scratch_shapes=[
            pltpu.VMEM((L, HA, BN), jnp.float32),
            pltpu.VMEM((L, HA, BN), jnp.float32),
            pltpu.VMEM((HIDDEN, BN), jnp.float32),
            pltpu.VMEM((N_VARS, N_VARS), jnp.float32),
        ],
        compiler_params=pltpu.CompilerParams(
            vmem_limit_bytes=64 * 1024 * 1024,
        ),
    )(xa, adjacency_matrix, W0a, WxrTa, WhT, mpWT, mpbT, WoutT, boutT)

    return out.reshape(HORIZON, B, N_VARS).transpose(1, 0, 2)


# speedup vs baseline: 1.1288x; 1.0013x over previous
import functools

import jax
import jax.numpy as jnp
from jax.experimental import pallas as pl
from jax.experimental.pallas import tpu as pltpu

N_VARS = 400
HIDDEN = 64
LAYERS = 4
HORIZON = 24
B = 4
L = 36
BN = B * N_VARS
HA = HIDDEN + 8


def _model_kernel(xa_ref, A_ref, W0a_ref, WxrTa_ref, WhT_ref,
                  mpWT_ref, mpbT_ref, WoutT_ref, boutT_ref,
                  out_ref, hsA_ref, hsB_ref, hT_ref, adj_ref):
    f32 = jnp.float32
    H = HIDDEN

    a = A_ref[:, :]
    a = a - jnp.max(a, axis=1, keepdims=True)
    e = jnp.exp(a)
    adj_ref[:, :] = e / jnp.sum(e, axis=1, keepdims=True)

    ones_block = jnp.concatenate(
        [jnp.ones((1, BN), f32), jnp.zeros((7, BN), f32)], axis=0)
    for t in range(L):
        hsA_ref[t, H:, :] = ones_block
        hsB_ref[t, H:, :] = ones_block

    def sigmoid(v):
        return 0.5 * jnp.tanh(0.5 * v) + 0.5

    def gates(gx, gh, hv):
        r = sigmoid(gx[:H] + gh[:H])
        z = sigmoid(gx[H:2 * H] + gh[H:2 * H])
        n = jnp.tanh(gx[2 * H:] + r * gh[2 * H:])
        return n + z * (hv - n)

    def message_passing(l):
        adjv = adj_ref[:, :]
        mpWT_l = mpWT_ref[l]
        mpbT_l = mpbT_ref[l]
        for bb in range(B):
            sl = slice(bb * N_VARS, (bb + 1) * N_VARS)
            hb = hT_ref[:, sl]
            m = jnp.dot(hb, adjv, preferred_element_type=f32)
            mb = jnp.dot(mpWT_l, m, preferred_element_type=f32) + mpbT_l
            elu = jnp.where(mb > 0, mb, jnp.exp(mb) - 1.0)
            hT_ref[:, sl] = elu + hb

    CHUNKS = ((0, 768), (768, 1600))

    W0a = W0a_ref[:, :]
    WhT0 = WhT_ref[0]
    hvs = [jnp.zeros((H, c1 - c0), dtype=f32) for c0, c1 in CHUNKS]
    for t in range(L):
        for i, (c0, c1) in enumerate(CHUNKS):
            gx = jnp.dot(W0a, xa_ref[t, :, c0:c1],
                         preferred_element_type=f32)
            gh = jnp.dot(WhT0, hvs[i], preferred_element_type=f32)
            hvs[i] = gates(gx, gh, hvs[i])
            hsA_ref[t, :H, c0:c1] = hvs[i]
    for i, (c0, c1) in enumerate(CHUNKS):
        hT_ref[:, c0:c1] = hvs[i]
    message_passing(0)

    for l in range(1, LAYERS):
        WxrTa_l = WxrTa_ref[l - 1]
        WhT_l = WhT_ref[l]
        src_ref = hsA_ref if l % 2 == 1 else hsB_ref
        dst_ref = hsB_ref if l % 2 == 1 else hsA_ref

        def step(t, hvt, WxrTa_l=WxrTa_l, WhT_l=WhT_l,
                 src_ref=src_ref, dst_ref=dst_ref):
            out = []
            for i, (c0, c1) in enumerate(CHUNKS):
                x_t = src_ref[t, :, c0:c1]
                gx = jnp.dot(WxrTa_l, x_t, preferred_element_type=f32)
                gh = jnp.dot(WhT_l, hvt[i], preferred_element_type=f32)
                hn = gates(gx, gh, hvt[i])
                dst_ref[t, :H, c0:c1] = hn
                out.append(hn)
            return tuple(out)

        hvt = jax.lax.fori_loop(
            0, L, step,
            tuple(hT_ref[:, c0:c1] for c0, c1 in CHUNKS), unroll=18)
        for i, (c0, c1) in enumerate(CHUNKS):
            hT_ref[:, c0:c1] = hvt[i]
        message_passing(l)

    out_ref[:, :] = (jnp.dot(WoutT_ref[:, :], hT_ref[:, :],
                             preferred_element_type=f32) + boutT_ref[:, :])


@functools.partial(jax.jit, static_argnames=())
def kernel(x, adjacency_matrix, gru_Wx_first, gru_Wx_rest, gru_Wh, gru_b,
           mp_W, mp_b, W_out, b_out):
    f32 = jnp.float32
    xrows = x.transpose(1, 0, 2).reshape(L, 1, BN)
    xa = jnp.concatenate(
        [xrows,
         jnp.ones((L, 1, BN), f32),
         jnp.zeros((L, 6, BN), f32)], axis=1)
    W0a = jnp.concatenate(
        [gru_Wx_first.T, gru_b[0][:, None],
         jnp.zeros((3 * HIDDEN, 6), f32)], axis=1)
    WxrTa = jnp.concatenate(
        [gru_Wx_rest.transpose(0, 2, 1),
         gru_b[1:][:, :, None],
         jnp.zeros((LAYERS - 1, 3 * HIDDEN, 7), f32)], axis=2)
    WhT = gru_Wh.transpose(0, 2, 1)
    mpWT = mp_W.transpose(0, 2, 1)
    mpbT = mp_b[:, :, None]
    WoutT = W_out.T
    boutT = b_out[:, None]

    out = pl.pallas_call(
        _model_kernel,
        out_shape=jax.ShapeDtypeStruct((HORIZON, BN), jnp.float32),
        scratch_shapes=[
            pltpu.VMEM((L, HA, BN), jnp.float32),
            pltpu.VMEM((L, HA, BN), jnp.float32),
            pltpu.VMEM((HIDDEN, BN), jnp.float32),
            pltpu.VMEM((N_VARS, N_VARS), jnp.float32),
        ],
        compiler_params=pltpu.CompilerParams(
            vmem_limit_bytes=64 * 1024 * 1024,
        ),
    )(xa, adjacency_matrix, W0a, WxrTa, WhT, mpWT, mpbT, WoutT, boutT)

    return out.reshape(HORIZON, B, N_VARS).transpose(1, 0, 2)


# speedup vs baseline: 1.1288x; 1.0000x over previous
import functools

import jax
import jax.numpy as jnp
from jax.experimental import pallas as pl
from jax.experimental.pallas import tpu as pltpu

N_VARS = 400
HIDDEN = 64
LAYERS = 4
HORIZON = 24
B = 4
L = 36
BN = B * N_VARS
HA = HIDDEN + 8


def _model_kernel(xa_ref, A_ref, W0a_ref, WxrTa_ref, WhT_ref,
                  mpWT_ref, mpbT_ref, WoutT_ref, boutT_ref,
                  out_ref, hsA_ref, hsB_ref, hT_ref, adj_ref):
    f32 = jnp.float32
    H = HIDDEN

    a = A_ref[:, :]
    a = a - jnp.max(a, axis=1, keepdims=True)
    e = jnp.exp(a)
    adj_ref[:, :] = e / jnp.sum(e, axis=1, keepdims=True)

    ones_block = jnp.concatenate(
        [jnp.ones((1, BN), f32), jnp.zeros((7, BN), f32)], axis=0)
    for t in range(L):
        hsA_ref[t, H:, :] = ones_block
        hsB_ref[t, H:, :] = ones_block

    def sigmoid(v):
        return 0.5 * jnp.tanh(0.5 * v) + 0.5

    def gates(gx, gh, hv):
        r = sigmoid(gx[:H] + gh[:H])
        z = sigmoid(gx[H:2 * H] + gh[H:2 * H])
        n = jnp.tanh(gx[2 * H:] + r * gh[2 * H:])
        return n + z * (hv - n)

    def message_passing(l):
        adjv = adj_ref[:, :]
        mpWT_l = mpWT_ref[l]
        mpbT_l = mpbT_ref[l]
        for bb in range(B):
            sl = slice(bb * N_VARS, (bb + 1) * N_VARS)
            hb = hT_ref[:, sl]
            m = jnp.dot(hb, adjv, preferred_element_type=f32)
            mb = jnp.dot(mpWT_l, m, preferred_element_type=f32) + mpbT_l
            elu = jnp.where(mb > 0, mb, jnp.exp(mb) - 1.0)
            hT_ref[:, sl] = elu + hb

    CHUNKS = ((0, 768), (768, 1600))

    W0a = W0a_ref[:, :]
    WhT0 = WhT_ref[0]
    hvs = [jnp.zeros((H, c1 - c0), dtype=f32) for c0, c1 in CHUNKS]
    for t in range(L):
        for i, (c0, c1) in enumerate(CHUNKS):
            gx = jnp.dot(W0a, xa_ref[t, :, c0:c1],
                         preferred_element_type=f32)
            gh = jnp.dot(WhT0, hvs[i], preferred_element_type=f32)
            hvs[i] = gates(gx, gh, hvs[i])
            hsA_ref[t, :H, c0:c1] = hvs[i]
    for i, (c0, c1) in enumerate(CHUNKS):
        hT_ref[:, c0:c1] = hvs[i]
    message_passing(0)

    for l in range(1, LAYERS):
        WxrTa_l = WxrTa_ref[l - 1]
        WhT_l = WhT_ref[l]
        src_ref = hsA_ref if l % 2 == 1 else hsB_ref
        dst_ref = hsB_ref if l % 2 == 1 else hsA_ref

        def step(t, hvt, WxrTa_l=WxrTa_l, WhT_l=WhT_l,
                 src_ref=src_ref, dst_ref=dst_ref):
            out = []
            for i, (c0, c1) in enumerate(CHUNKS):
                x_t = src_ref[t, :, c0:c1]
                gx = jnp.dot(WxrTa_l, x_t, preferred_element_type=f32)
                gh = jnp.dot(WhT_l, hvt[i], preferred_element_type=f32)
                hn = gates(gx, gh, hvt[i])
                dst_ref[t, :H, c0:c1] = hn
                out.append(hn)
            return tuple(out)

        hvt = jax.lax.fori_loop(
            0, L, step,
            tuple(hT_ref[:, c0:c1] for c0, c1 in CHUNKS), unroll=9)
        for i, (c0, c1) in enumerate(CHUNKS):
            hT_ref[:, c0:c1] = hvt[i]
        message_passing(l)

    out_ref[:, :] = (jnp.dot(WoutT_ref[:, :], hT_ref[:, :],
                             preferred_element_type=f32) + boutT_ref[:, :])


@functools.partial(jax.jit, static_argnames=())
def kernel(x, adjacency_matrix, gru_Wx_first, gru_Wx_rest, gru_Wh, gru_b,
           mp_W, mp_b, W_out, b_out):
    f32 = jnp.float32
    xrows = x.transpose(1, 0, 2).reshape(L, 1, BN)
    xa = jnp.concatenate(
        [xrows,
         jnp.ones((L, 1, BN), f32),
         jnp.zeros((L, 6, BN), f32)], axis=1)
    W0a = jnp.concatenate(
        [gru_Wx_first.T, gru_b[0][:, None],
         jnp.zeros((3 * HIDDEN, 6), f32)], axis=1)
    WxrTa = jnp.concatenate(
        [gru_Wx_rest.transpose(0, 2, 1),
         gru_b[1:][:, :, None],
         jnp.zeros((LAYERS - 1, 3 * HIDDEN, 7), f32)], axis=2)
    WhT = gru_Wh.transpose(0, 2, 1)
    mpWT = mp_W.transpose(0, 2, 1)
    mpbT = mp_b[:, :, None]
    WoutT = W_out.T
    boutT = b_out[:, None]

    out = pl.pallas_call(
        _model_kernel,
        out_shape=jax.ShapeDtypeStruct((HORIZON, BN), jnp.float32),
        scratch_shapes=[
            pltpu.VMEM((L, HA, BN), jnp.float32),
            pltpu.VMEM((L, HA, BN), jnp.float32),
            pltpu.VMEM((HIDDEN, BN), jnp.float32),
            pltpu.VMEM((N_VARS, N_VARS), jnp.float32),
        ],
        compiler_params=pltpu.CompilerParams(
            vmem_limit_bytes=64 * 1024 * 1024,
        ),
    )(xa, adjacency_matrix, W0a, WxrTa, WhT, mpWT, mpbT, WoutT, boutT)

    return out.reshape(HORIZON, B, N_VARS).transpose(1, 0, 2)
